# Initial kernel scaffold; baseline (speedup 1.0000x reference)
import jax
import jax.numpy as jnp
from jax.experimental import pallas as pl


def kernel(X, adj_mat, W_rel0, b_rel0, W_root0, W_rel1, b_rel1, W_root1, W_rel2, b_rel2, W_root2):
    raise NotImplementedError("write your pallas kernel here")


# speedup vs baseline: 1.2828x; 1.2828x over previous
import functools

import jax
import jax.numpy as jnp
from jax.experimental import pallas as pl
from jax.experimental.pallas import tpu as pltpu

_BI = 256


def _fused_body(x_ref, adj_ref,
                wr0_ref, br0_ref, wt0_ref,
                wr1_ref, br1_ref, wt1_ref,
                wr2_ref, br2_ref, wt2_ref,
                out_ref,
                adjbf_ref, ha_ref, hb_ref, hwt_ref, hwtbf_ref):
    l = pl.program_id(0)
    i = pl.program_id(1)

    @pl.when(jnp.logical_and(l == 0, i == 0))
    def _():
        hw = jnp.dot(x_ref[...], wr0_ref[...].T,
                     preferred_element_type=jnp.float32)
        hwt_ref[...] = hw.T

    @pl.when(jnp.logical_and(l == 1, i == 0))
    def _():
        hw = jnp.dot(ha_ref[...], wr1_ref[...].T,
                     preferred_element_type=jnp.float32)
        hwtbf_ref[...] = hw.T.astype(jnp.bfloat16)

    @pl.when(jnp.logical_and(l == 2, i == 0))
    def _():
        hw = jnp.dot(hb_ref[...], wr2_ref[...].T,
                     preferred_element_type=jnp.float32)
        hwtbf_ref[...] = hw.T.astype(jnp.bfloat16)

    sl = pl.ds(i * _BI, _BI)

    @pl.when(l == 0)
    def _():
        slab = adj_ref[...]
        adjbf_ref[i] = slab.astype(jnp.bfloat16)
        agg_t = jnp.dot(hwt_ref[...], slab, preferred_element_type=jnp.float32)
        root = jnp.dot(x_ref[sl, :], wt0_ref[...].T,
                       preferred_element_type=jnp.float32)
        ha_ref[sl, :] = jnp.maximum(agg_t.T + root + br0_ref[...], 0.0)

    @pl.when(l == 1)
    def _():
        agg_t = jnp.dot(hwtbf_ref[...], adjbf_ref[i],
                        preferred_element_type=jnp.float32)
        root = jnp.dot(ha_ref[sl, :], wt1_ref[...].T,
                       preferred_element_type=jnp.float32)
        hb_ref[sl, :] = jnp.maximum(agg_t.T + root + br1_ref[...], 0.0)

    @pl.when(l == 2)
    def _():
        agg_t = jnp.dot(hwtbf_ref[...], adjbf_ref[i],
                        preferred_element_type=jnp.float32)
        root = jnp.dot(hb_ref[sl, :], wt2_ref[...].T,
                       preferred_element_type=jnp.float32)
        out_ref[...] = jnp.maximum(agg_t.T + root + br2_ref[...], 0.0)


def kernel(X, adj_mat, W_rel0, b_rel0, W_root0, W_rel1, b_rel1, W_root1,
           W_rel2, b_rel2, W_root2):
    n, d = X.shape
    h = W_rel0.shape[0]
    ni = n // _BI

    full = lambda shape: pl.BlockSpec(shape, lambda l, i: (0, 0))
    out = pl.pallas_call(
        _fused_body,
        grid=(3, ni),
        in_specs=[
            full((n, d)),
            pl.BlockSpec((n, _BI),
                         lambda l, i: (0, jnp.where(l == 0, i, ni - 1))),
            full((h, d)), full((1, h)), full((h, d)),
            full((h, h)), full((1, h)), full((h, h)),
            full((h, h)), full((1, h)), full((h, h)),
        ],
        out_specs=pl.BlockSpec((_BI, h), lambda l, i: (i, 0)),
        out_shape=jax.ShapeDtypeStruct((n, h), jnp.float32),
        scratch_shapes=[
            pltpu.VMEM((ni, n, _BI), jnp.bfloat16),
            pltpu.VMEM((n, h), jnp.float32),
            pltpu.VMEM((n, h), jnp.float32),
            pltpu.VMEM((h, n), jnp.float32),
            pltpu.VMEM((h, n), jnp.bfloat16),
        ],
        compiler_params=pltpu.CompilerParams(
            vmem_limit_bytes=60 * 1024 * 1024,
        ),
    )(X, adj_mat,
      W_rel0, b_rel0.reshape(1, h), W_root0,
      W_rel1, b_rel1.reshape(1, h), W_root1,
      W_rel2, b_rel2.reshape(1, h), W_root2)
    return out
